# Initial kernel scaffold; baseline (speedup 1.0000x reference)
import jax
import jax.numpy as jnp
from jax.experimental import pallas as pl


def kernel(inputs, cluster_centers):
    raise NotImplementedError("write your pallas kernel here")


# speedup vs baseline: 1.9763x; 1.9763x over previous
import jax
import jax.numpy as jnp
from jax.experimental import pallas as pl

BATCH = 4096
N_CLUSTERS = 8192
B_TILE = 512


def _assign_kernel(x_ref, c_ref, out_ref):
    x0 = x_ref[:, 0:1]
    x1 = x_ref[:, 1:2]
    c0 = c_ref[0:1, :]
    c1 = c_ref[1:2, :]
    d0 = x0 - c0
    d1 = x1 - c1
    dist = d0 * d0 + d1 * d1
    out_ref[:] = jnp.argmin(dist, axis=-1).astype(jnp.int32)


def kernel(inputs, cluster_centers):
    centers_t = cluster_centers.T
    grid = (BATCH // B_TILE,)
    return pl.pallas_call(
        _assign_kernel,
        grid=grid,
        in_specs=[
            pl.BlockSpec((B_TILE, 2), lambda i: (i, 0)),
            pl.BlockSpec((2, N_CLUSTERS), lambda i: (0, 0)),
        ],
        out_specs=pl.BlockSpec((B_TILE,), lambda i: (i,)),
        out_shape=jax.ShapeDtypeStruct((BATCH,), jnp.int32),
    )(inputs, centers_t)
